# Initial kernel scaffold; baseline (speedup 1.0000x reference)
import jax
import jax.numpy as jnp
from jax.experimental import pallas as pl


def kernel(multi_bboxes, multi_scores):
    raise NotImplementedError("write your pallas kernel here")


# speedup vs baseline: 3.7554x; 3.7554x over previous
import functools

import jax
import jax.numpy as jnp
import numpy as np
from jax import lax
from jax.experimental import pallas as pl
from jax.experimental.pallas import tpu as pltpu
from jax.experimental.pallas import tpu_sc as plsc

SCORE_THR = 0.05
MAX_NUM = 100
N = 5000
C = 20
L = 16
NP = 5008
NV = NP // L
KCAP = 128
OPAD = 128
NCORES = 2
NSUB = 16
BIG = np.int32(1 << 30)

_mesh = plsc.VectorSubcoreMesh(
    core_axis_name="c", subcore_axis_name="s",
    num_cores=NCORES, num_subcores=NSUB)

_f32 = np.float32
_i32 = np.int32


def _wid():
    return lax.axis_index("s") * NCORES + lax.axis_index("c")


def _sload(ref, idx):
    return ref[pl.ds(idx, L)][0]


def _sstore(ref, idx, val, lane0):
    plsc.store_scatter(
        ref, [jnp.full((L,), idx, _i32)], jnp.full((L,), val), mask=lane0)


@functools.partial(
    pl.kernel,
    out_type=jax.ShapeDtypeStruct((C * L,), _f32),
    mesh=_mesh,
    compiler_params=pltpu.CompilerParams(needs_layout_passes=False, use_tc_tiling_on_sc=False),
    scratch_types=[
        pltpu.VMEM((NP,), _f32),
        pltpu.VMEM((L,), _f32),
    ],
)
def _max_kernel(x2_hbm, y2_hbm, out_hbm, buf, mv):
    wid = _wid()

    @pl.when(wid < C)
    def _():
        def body(j, acc):
            return jnp.maximum(acc, buf[pl.ds(j * L, L)])

        pltpu.sync_copy(x2_hbm.at[wid], buf)
        acc = lax.fori_loop(0, NV, body, jnp.full((L,), -1e30, _f32))
        pltpu.sync_copy(y2_hbm.at[wid], buf)
        acc = lax.fori_loop(0, NV, body, acc)
        mv[...] = acc
        pltpu.sync_copy(mv, out_hbm.at[pl.ds(wid * L, L)])


@functools.partial(
    pl.kernel,
    out_type=[jax.ShapeDtypeStruct((C, KCAP), _f32)] * 5
    + [jax.ShapeDtypeStruct((C, KCAP), _i32)],
    mesh=_mesh,
    compiler_params=pltpu.CompilerParams(needs_layout_passes=False, use_tc_tiling_on_sc=False),
    scratch_types=[
        pltpu.VMEM((NP,), _f32),
        pltpu.VMEM((NP + L,), _f32),
        pltpu.VMEM((NP + L,), _f32),
        pltpu.VMEM((NP + L,), _f32),
        pltpu.VMEM((NP + L,), _f32),
        pltpu.VMEM((NP + L,), _f32),
        pltpu.VMEM((NP + L,), _f32),
        pltpu.VMEM((NP + L,), _f32),
        pltpu.VMEM((NP + L,), _f32),
        pltpu.VMEM((NP + L,), _f32),
        pltpu.VMEM((C * L,), _f32),
        pltpu.VMEM((KCAP,), _f32),
        pltpu.VMEM((KCAP,), _f32),
        pltpu.VMEM((KCAP,), _f32),
        pltpu.VMEM((KCAP,), _f32),
        pltpu.VMEM((KCAP,), _f32),
        pltpu.VMEM((KCAP,), _i32),
    ],
)
def _nms_kernel(s_hbm, x1_hbm, y1_hbm, x2_hbm, y2_hbm, maxv_hbm,
                osc_hbm, ox1_hbm, oy1_hbm, ox2_hbm, oy2_hbm, on_hbm,
                sv, gx1, gy1, gx2, gy2, ox1, oy1, ox2, oy2, oar,
                mbuf, ks, kx1, ky1, kx2, ky2, kn):
    wid = _wid()
    iota = lax.iota(_i32, L)
    lane0 = iota == 0

    @pl.when(wid < C)
    def _():
        pltpu.sync_copy(maxv_hbm, mbuf)
        pltpu.sync_copy(s_hbm.at[wid], sv)
        pltpu.sync_copy(x1_hbm.at[wid], gx1.at[pl.ds(0, NP)])
        pltpu.sync_copy(y1_hbm.at[wid], gy1.at[pl.ds(0, NP)])
        pltpu.sync_copy(x2_hbm.at[wid], gx2.at[pl.ds(0, NP)])
        pltpu.sync_copy(y2_hbm.at[wid], gy2.at[pl.ds(0, NP)])

        def mbody(j, acc):
            return jnp.maximum(acc, mbuf[pl.ds(j * L, L)])

        mxv = lax.fori_loop(0, C, mbody, jnp.full((L,), -1e30, _f32))
        gmax = jnp.max(mxv)
        off = wid.astype(_f32) * (gmax + _f32(1.0))

        def p1(j, carry):
            bv, bi = carry
            sl = pl.ds(j * L, L)
            s = sv[sl]
            s = jnp.where(s > _f32(SCORE_THR), s, _f32(-1.0))
            sv[sl] = s
            x1 = gx1[sl] + off
            y1 = gy1[sl] + off
            x2 = gx2[sl] + off
            y2 = gy2[sl] + off
            ox1[sl] = x1
            oy1[sl] = y1
            ox2[sl] = x2
            oy2[sl] = y2
            oar[sl] = (x2 - x1) * (y2 - y1)
            idx = j * L + iota
            upd = s > bv
            return jnp.where(upd, s, bv), jnp.where(upd, idx, bi)

        bv0, bi0 = lax.fori_loop(
            0, NV, p1,
            (jnp.full((L,), -2.0, _f32), jnp.zeros((L,), _i32)))

        def ki(j, _):
            sl = pl.ds(j * L, L)
            ks[sl] = jnp.full((L,), -1.0, _f32)
            kx1[sl] = jnp.zeros((L,), _f32)
            ky1[sl] = jnp.zeros((L,), _f32)
            kx2[sl] = jnp.zeros((L,), _f32)
            ky2[sl] = jnp.zeros((L,), _f32)
            kn[sl] = jnp.zeros((L,), _i32)
            return 0

        lax.fori_loop(0, KCAP // L, ki, 0)

        def cond(carry):
            cnt, m, _, _ = carry
            return (cnt < MAX_NUM) & (m > _f32(0.0))

        def body(carry):
            cnt, m, bv, bi = carry
            cand = jnp.where(bv == m, bi, BIG)
            nsel = jnp.min(cand)
            _sstore(ks, cnt, m, lane0)
            _sstore(kx1, cnt, _sload(gx1, nsel), lane0)
            _sstore(ky1, cnt, _sload(gy1, nsel), lane0)
            _sstore(kx2, cnt, _sload(gx2, nsel), lane0)
            _sstore(ky2, cnt, _sload(gy2, nsel), lane0)
            _sstore(kn, cnt, nsel, lane0)
            bx1 = _sload(ox1, nsel)
            by1 = _sload(oy1, nsel)
            bx2 = _sload(ox2, nsel)
            by2 = _sload(oy2, nsel)
            bar = _sload(oar, nsel)

            def sup(j, carry2):
                bv2, bi2 = carry2
                sl = pl.ds(j * L, L)
                s = sv[sl]
                ix1 = jnp.maximum(bx1, ox1[sl])
                iy1 = jnp.maximum(by1, oy1[sl])
                ix2 = jnp.minimum(bx2, ox2[sl])
                iy2 = jnp.minimum(by2, oy2[sl])
                inter = (jnp.maximum(ix2 - ix1, _f32(0.0))
                         * jnp.maximum(iy2 - iy1, _f32(0.0)))
                den = (bar + oar[sl]) - inter + _f32(1e-9)
                s = jnp.where(_f32(2.0) * inter > den, _f32(-1.0), s)
                sv[sl] = s
                idx = j * L + iota
                upd = s > bv2
                return jnp.where(upd, s, bv2), jnp.where(upd, idx, bi2)

            bv2, bi2 = lax.fori_loop(
                0, NV, sup,
                (jnp.full((L,), -2.0, _f32), jnp.zeros((L,), _i32)))
            return cnt + 1, jnp.max(bv2), bv2, bi2

        lax.while_loop(cond, body, (_i32(0), jnp.max(bv0), bv0, bi0))

        pltpu.sync_copy(ks, osc_hbm.at[wid])
        pltpu.sync_copy(kx1, ox1_hbm.at[wid])
        pltpu.sync_copy(ky1, oy1_hbm.at[wid])
        pltpu.sync_copy(kx2, ox2_hbm.at[wid])
        pltpu.sync_copy(ky2, oy2_hbm.at[wid])
        pltpu.sync_copy(kn, on_hbm.at[wid])


@functools.partial(
    pl.kernel,
    out_type=[jax.ShapeDtypeStruct((OPAD * 5,), _f32),
              jax.ShapeDtypeStruct((OPAD,), _i32)],
    mesh=_mesh,
    compiler_params=pltpu.CompilerParams(needs_layout_passes=False, use_tc_tiling_on_sc=False),
    scratch_types=[
        pltpu.VMEM((C * KCAP + L,), _f32),
        pltpu.VMEM((C * KCAP + L,), _f32),
        pltpu.VMEM((C * KCAP + L,), _f32),
        pltpu.VMEM((C * KCAP + L,), _f32),
        pltpu.VMEM((C * KCAP + L,), _f32),
        pltpu.VMEM((C * KCAP + L,), _i32),
        pltpu.VMEM((2 * L,), _f32),
        pltpu.VMEM((2 * L,), _i32),
        pltpu.SMEM((2 * L,), _i32),
        pltpu.VMEM((OPAD * 5,), _f32),
        pltpu.VMEM((OPAD,), _i32),
    ],
)
def _merge_kernel(sc_hbm, x1_hbm, y1_hbm, x2_hbm, y2_hbm, n_hbm,
                  dets_hbm, labels_hbm,
                  vsc, vx1, vy1, vx2, vy2, vn, hs, hn, hp, dv, lv):
    wid = _wid()
    iota = lax.iota(_i32, L)
    lane0 = iota == 0

    @pl.when(wid == 0)
    def _():
        pltpu.sync_copy(sc_hbm, vsc.at[pl.ds(0, C * KCAP)])
        pltpu.sync_copy(x1_hbm, vx1.at[pl.ds(0, C * KCAP)])
        pltpu.sync_copy(y1_hbm, vy1.at[pl.ds(0, C * KCAP)])
        pltpu.sync_copy(x2_hbm, vx2.at[pl.ds(0, C * KCAP)])
        pltpu.sync_copy(y2_hbm, vy2.at[pl.ds(0, C * KCAP)])
        pltpu.sync_copy(n_hbm, vn.at[pl.ds(0, C * KCAP)])

        for half in range(2):
            cv = iota + half * L
            cidx = jnp.minimum(cv, C - 1) * KCAP
            h = plsc.load_gather(vsc, [cidx])
            hs[pl.ds(half * L, L)] = jnp.where(cv < C, h, _f32(-1.0))
            nh = plsc.load_gather(vn, [cidx])
            hn[pl.ds(half * L, L)] = jnp.where(cv < C, nh, _i32(0))

        def pinit(c, _):
            hp[c] = _i32(0)
            return 0

        lax.fori_loop(0, 2 * L, pinit, 0)

        def oinit(k, _):
            lv[pl.ds(k * L, L)] = jnp.full((L,), -1, _i32)
            return 0

        lax.fori_loop(0, OPAD // L, oinit, 0)

        def zinit(k, _):
            dv[pl.ds(k * L, L)] = jnp.zeros((L,), _f32)
            return 0

        lax.fori_loop(0, OPAD * 5 // L, zinit, 0)

        def mbody(k, _):
            h1 = hs[pl.ds(0, L)]
            h2 = hs[pl.ds(L, L)]
            m = jnp.maximum(jnp.max(h1), jnp.max(h2))

            @pl.when(m > _f32(0.0))
            def _():
                n1 = hn[pl.ds(0, L)]
                n2 = hn[pl.ds(L, L)]
                fi1 = jnp.where(h1 == m, n1 * C + iota, BIG)
                fi2 = jnp.where(h2 == m, n2 * C + (iota + L), BIG)
                fi = jnp.minimum(jnp.min(fi1), jnp.min(fi2))
                csel = lax.rem(fi, _i32(C))
                p = hp[csel]
                base = csel * KCAP + p
                _sstore(dv, k * 5 + 0, _sload(vx1, base), lane0)
                _sstore(dv, k * 5 + 1, _sload(vy1, base), lane0)
                _sstore(dv, k * 5 + 2, _sload(vx2, base), lane0)
                _sstore(dv, k * 5 + 3, _sload(vy2, base), lane0)
                _sstore(dv, k * 5 + 4, m, lane0)
                _sstore(lv, k, csel, lane0)
                pn = p + 1
                hp[csel] = pn
                pc = jnp.minimum(pn, KCAP - 1)
                nxt = _sload(vsc, csel * KCAP + pc)
                _sstore(hs, csel, jnp.where(pn > KCAP - 1, _f32(-1.0), nxt),
                        lane0)
                _sstore(hn, csel, _sload(vn, csel * KCAP + pc), lane0)

            return 0

        lax.fori_loop(0, MAX_NUM, mbody, 0)

        pltpu.sync_copy(dv, dets_hbm)
        pltpu.sync_copy(lv, labels_hbm)


def kernel(multi_bboxes, multi_scores):
    bb = multi_bboxes.reshape(N, C, 4)
    sc = multi_scores[:, :C]
    pad = ((0, 0), (0, NP - N))
    x1t = jnp.pad(bb[:, :, 0].T, pad)
    y1t = jnp.pad(bb[:, :, 1].T, pad)
    x2t = jnp.pad(bb[:, :, 2].T, pad)
    y2t = jnp.pad(bb[:, :, 3].T, pad)
    st = jnp.pad(sc.T, pad)
    maxv = _max_kernel(x2t, y2t)
    k_sc, k_x1, k_y1, k_x2, k_y2, k_n = _nms_kernel(
        st, x1t, y1t, x2t, y2t, maxv)
    dets_pad, labels_pad = _merge_kernel(
        k_sc.reshape(-1), k_x1.reshape(-1), k_y1.reshape(-1),
        k_x2.reshape(-1), k_y2.reshape(-1), k_n.reshape(-1))
    dets = dets_pad.reshape(OPAD, 5)[:MAX_NUM]
    labels = labels_pad[:MAX_NUM]
    return dets, labels


# speedup vs baseline: 17.4624x; 4.6499x over previous
import functools

import jax
import jax.numpy as jnp
import numpy as np
from jax import lax
from jax.experimental import pallas as pl
from jax.experimental.pallas import tpu as pltpu
from jax.experimental.pallas import tpu_sc as plsc

SCORE_THR = 0.05
MAX_NUM = 100
N = 5000
C = 20
L = 16
NP = 5008
NV = NP // L
KCAP = 128
OPAD = 128
NCORES = 2
NSUB = 16
BIG = np.int32(1 << 30)

_mesh = plsc.VectorSubcoreMesh(
    core_axis_name="c", subcore_axis_name="s",
    num_cores=NCORES, num_subcores=NSUB)

_f32 = np.float32
_i32 = np.int32


def _wid():
    return lax.axis_index("s") * NCORES + lax.axis_index("c")


def _sload(ref, idx):
    return ref[pl.ds(idx, L)][0]


def _sstore(ref, idx, val, lane0):
    plsc.store_scatter(
        ref, [jnp.full((L,), idx, _i32)], jnp.full((L,), val), mask=lane0)


@functools.partial(
    pl.kernel,
    out_type=jax.ShapeDtypeStruct((C * L,), _f32),
    mesh=_mesh,
    compiler_params=pltpu.CompilerParams(needs_layout_passes=False, use_tc_tiling_on_sc=False),
    scratch_types=[
        pltpu.VMEM((NP,), _f32),
        pltpu.VMEM((L,), _f32),
    ],
)
def _max_kernel(x2_hbm, y2_hbm, out_hbm, buf, mv):
    wid = _wid()

    @pl.when(wid < C)
    def _():
        def body(j, acc):
            return jnp.maximum(acc, buf[pl.ds(j * L, L)])

        pltpu.sync_copy(x2_hbm.at[wid], buf)
        acc = lax.fori_loop(0, NV, body, jnp.full((L,), -1e30, _f32))
        pltpu.sync_copy(y2_hbm.at[wid], buf)
        acc = lax.fori_loop(0, NV, body, acc)
        mv[...] = acc
        pltpu.sync_copy(mv, out_hbm.at[pl.ds(wid * L, L)])


NB = 512
CH = 256


@functools.partial(
    pl.kernel,
    out_type=[jax.ShapeDtypeStruct((C, KCAP), _f32)] * 5
    + [jax.ShapeDtypeStruct((C, KCAP), _i32)],
    mesh=_mesh,
    compiler_params=pltpu.CompilerParams(needs_layout_passes=False, use_tc_tiling_on_sc=False),
    scratch_types=[
        pltpu.VMEM((NP,), _f32),
        pltpu.VMEM((NP + L,), _f32),
        pltpu.VMEM((NP + L,), _f32),
        pltpu.VMEM((NP + L,), _f32),
        pltpu.VMEM((NP + L,), _f32),
        pltpu.VMEM((NP + L,), _f32),
        pltpu.VMEM((NP + L,), _f32),
        pltpu.VMEM((NP + L,), _f32),
        pltpu.VMEM((NP + L,), _f32),
        pltpu.VMEM((NP + L,), _f32),
        pltpu.VMEM((NP,), _i32),
        pltpu.VMEM((NB * L,), _i32),
        pltpu.VMEM((C * L,), _f32),
        pltpu.VMEM((NP + L,), _f32),
        pltpu.VMEM((NP + L,), _i32),
        pltpu.VMEM((NP + L,), _f32),
        pltpu.VMEM((NP + L,), _f32),
        pltpu.VMEM((NP + L,), _f32),
        pltpu.VMEM((NP + L,), _f32),
        pltpu.VMEM((NP + L,), _f32),
        pltpu.VMEM((KCAP,), _f32),
        pltpu.VMEM((KCAP,), _f32),
        pltpu.VMEM((KCAP,), _f32),
        pltpu.VMEM((KCAP,), _f32),
        pltpu.VMEM((KCAP,), _f32),
        pltpu.VMEM((KCAP,), _i32),
        pltpu.VMEM((KCAP,), _f32),
        pltpu.VMEM((KCAP,), _f32),
        pltpu.VMEM((KCAP,), _f32),
        pltpu.VMEM((KCAP,), _f32),
        pltpu.VMEM((KCAP,), _f32),
    ],
)
def _nms_kernel(s_hbm, x1_hbm, y1_hbm, x2_hbm, y2_hbm, maxv_hbm,
                osc_hbm, ox1_hbm, oy1_hbm, ox2_hbm, oy2_hbm, on_hbm,
                sv, gx1, gy1, gx2, gy2, ox1, oy1, ox2, oy2, oar,
                bkt, hist, mbuf, cs, cidx, cx1, cy1, cx2, cy2, car,
                ks, kx1, ky1, kx2, ky2, kn, kbx1, kby1, kbx2, kby2, kbar):
    wid = _wid()
    iota = lax.iota(_i32, L)
    lane0 = iota == 0
    ones = jnp.ones((L,), _i32)

    @pl.when(wid < C)
    def _():
        pltpu.sync_copy(maxv_hbm, mbuf)
        pltpu.sync_copy(s_hbm.at[wid], sv)
        pltpu.sync_copy(x1_hbm.at[wid], gx1.at[pl.ds(0, NP)])
        pltpu.sync_copy(y1_hbm.at[wid], gy1.at[pl.ds(0, NP)])
        pltpu.sync_copy(x2_hbm.at[wid], gx2.at[pl.ds(0, NP)])
        pltpu.sync_copy(y2_hbm.at[wid], gy2.at[pl.ds(0, NP)])

        def mbody(j, acc):
            return jnp.maximum(acc, mbuf[pl.ds(j * L, L)])

        mxv = lax.fori_loop(0, C, mbody, jnp.full((L,), -1e30, _f32))
        gmax = jnp.max(mxv)
        off = wid.astype(_f32) * (gmax + _f32(1.0))

        def hz(j, _):
            hist[pl.ds(j * L, L)] = jnp.zeros((L,), _i32)
            return 0

        lax.fori_loop(0, NB, hz, 0)

        def p1(j, vcnt):
            sl = pl.ds(j * L, L)
            s = sv[sl]
            s = jnp.where(s > _f32(SCORE_THR), s, _f32(-1.0))
            sv[sl] = s
            x1 = gx1[sl] + off
            y1 = gy1[sl] + off
            x2 = gx2[sl] + off
            y2 = gy2[sl] + off
            ox1[sl] = x1
            oy1[sl] = y1
            ox2[sl] = x2
            oy2[sl] = y2
            oar[sl] = (x2 - x1) * (y2 - y1)
            valid = s > _f32(0.0)
            b = jnp.clip((s * _f32(NB)).astype(_i32), 0, NB - 1)
            bkt[sl] = jnp.where(valid, b, -1)
            plsc.addupdate_scatter(hist, [b * L + iota], ones, mask=valid)
            return vcnt + plsc.all_reduce_population_count(valid)[0]

        vcnt = lax.fori_loop(0, NV, p1, _i32(0))

        def ki(j, _):
            sl = pl.ds(j * L, L)
            ks[sl] = jnp.full((L,), -1.0, _f32)
            kx1[sl] = jnp.zeros((L,), _f32)
            ky1[sl] = jnp.zeros((L,), _f32)
            kx2[sl] = jnp.zeros((L,), _f32)
            ky2[sl] = jnp.zeros((L,), _f32)
            kn[sl] = jnp.zeros((L,), _i32)
            return 0

        lax.fori_loop(0, KCAP // L, ki, 0)

        def outer_cond(st):
            cnt, bp, rem = st
            return (cnt < MAX_NUM) & (rem > 0) & (bp >= 0)

        def outer_body(st):
            cnt, bp, rem = st

            def wcond(ws):
                acc, bptr = ws
                return (acc < CH) & (bptr >= 0)

            def wbody(ws):
                acc, bptr = ws
                cb = jnp.sum(hist[pl.ds(bptr * L, L)])
                return acc + cb, bptr - 1

            acc, bptr = lax.while_loop(wcond, wbody, (_i32(0), bp))
            b_lo = bptr + 1
            rem = rem - acc

            def coll(j, wp):
                sl = pl.ds(j * L, L)
                b = bkt[sl]
                msk = (b >= b_lo) & (b <= bp)
                plsc.store_compressed(cs.at[pl.ds(wp, L)], sv[sl], mask=msk)
                plsc.store_compressed(cidx.at[pl.ds(wp, L)],
                                      j * L + iota, mask=msk)
                return wp + plsc.all_reduce_population_count(msk)[0]

            m_sz = lax.fori_loop(0, NV, coll, _i32(0))
            cs[pl.ds(m_sz, L)] = jnp.full((L,), -1.0, _f32)
            cidx[pl.ds(m_sz, L)] = jnp.zeros((L,), _i32)
            mv = lax.div(m_sz + (L - 1), _i32(L))

            def cg(j, _):
                sl = pl.ds(j * L, L)
                iv = cidx[sl]
                cx1[sl] = plsc.load_gather(ox1, [iv])
                cy1[sl] = plsc.load_gather(oy1, [iv])
                cx2[sl] = plsc.load_gather(ox2, [iv])
                cy2[sl] = plsc.load_gather(oy2, [iv])
                car[sl] = plsc.load_gather(oar, [iv])
                return 0

            lax.fori_loop(0, mv, cg, 0)

            def kchk(k, _):
                bx1 = _sload(kbx1, k)
                by1 = _sload(kby1, k)
                bx2 = _sload(kbx2, k)
                by2 = _sload(kby2, k)
                bar = _sload(kbar, k)

                def kchk_j(j, _2):
                    sl = pl.ds(j * L, L)
                    inter = (jnp.maximum(
                        jnp.minimum(bx2, cx2[sl]) - jnp.maximum(bx1, cx1[sl]),
                        _f32(0.0))
                        * jnp.maximum(
                        jnp.minimum(by2, cy2[sl]) - jnp.maximum(by1, cy1[sl]),
                        _f32(0.0)))
                    den = (bar + car[sl]) - inter + _f32(1e-9)
                    cs[sl] = jnp.where(_f32(2.0) * inter > den, _f32(-1.0),
                                       cs[sl])
                    return 0

                lax.fori_loop(0, mv, kchk_j, 0)
                return 0

            lax.fori_loop(0, cnt, kchk, 0)

            def am(j, carry):
                bv, bi = carry
                s = cs[pl.ds(j * L, L)]
                li = j * L + iota
                upd = s > bv
                return jnp.where(upd, s, bv), jnp.where(upd, li, bi)

            bv0, bi0 = lax.fori_loop(
                0, mv, am,
                (jnp.full((L,), -2.0, _f32), jnp.zeros((L,), _i32)))

            def cond(carry):
                cnt2, m, _, _ = carry
                return (cnt2 < MAX_NUM) & (m > _f32(0.0))

            def body(carry):
                cnt2, m, bv, bi = carry
                cand = jnp.where(bv == m, bi, BIG)
                lsel = jnp.full((L,), jnp.min(cand), _i32)
                nsel = plsc.load_gather(cidx, [lsel])
                cntv = jnp.full((L,), cnt2, _i32)
                mvz = jnp.full((L,), m, _f32)
                plsc.store_scatter(ks, [cntv], mvz, mask=lane0)
                plsc.store_scatter(kx1, [cntv],
                                   plsc.load_gather(gx1, [nsel]), mask=lane0)
                plsc.store_scatter(ky1, [cntv],
                                   plsc.load_gather(gy1, [nsel]), mask=lane0)
                plsc.store_scatter(kx2, [cntv],
                                   plsc.load_gather(gx2, [nsel]), mask=lane0)
                plsc.store_scatter(ky2, [cntv],
                                   plsc.load_gather(gy2, [nsel]), mask=lane0)
                plsc.store_scatter(kn, [cntv], nsel, mask=lane0)
                bx1 = plsc.load_gather(cx1, [lsel])
                by1 = plsc.load_gather(cy1, [lsel])
                bx2 = plsc.load_gather(cx2, [lsel])
                by2 = plsc.load_gather(cy2, [lsel])
                bar = plsc.load_gather(car, [lsel])
                plsc.store_scatter(kbx1, [cntv], bx1, mask=lane0)
                plsc.store_scatter(kby1, [cntv], by1, mask=lane0)
                plsc.store_scatter(kbx2, [cntv], bx2, mask=lane0)
                plsc.store_scatter(kby2, [cntv], by2, mask=lane0)
                plsc.store_scatter(kbar, [cntv], bar, mask=lane0)

                def sup(j, carry2):
                    bv2, bi2 = carry2
                    sl = pl.ds(j * L, L)
                    s = cs[sl]
                    inter = (jnp.maximum(
                        jnp.minimum(bx2, cx2[sl]) - jnp.maximum(bx1, cx1[sl]),
                        _f32(0.0))
                        * jnp.maximum(
                        jnp.minimum(by2, cy2[sl]) - jnp.maximum(by1, cy1[sl]),
                        _f32(0.0)))
                    den = (bar + car[sl]) - inter + _f32(1e-9)
                    s = jnp.where(_f32(2.0) * inter > den, _f32(-1.0), s)
                    cs[sl] = s
                    li = j * L + iota
                    upd = s > bv2
                    return (jnp.where(upd, s, bv2), jnp.where(upd, li, bi2))

                bv2, bi2 = lax.fori_loop(
                    0, mv, sup,
                    (jnp.full((L,), -2.0, _f32), jnp.zeros((L,), _i32)))
                return cnt2 + 1, jnp.max(bv2), bv2, bi2

            cnt, _, _, _ = lax.while_loop(
                cond, body, (cnt, jnp.max(bv0), bv0, bi0))
            return cnt, bptr, rem

        lax.while_loop(outer_cond, outer_body, (_i32(0), _i32(NB - 1), vcnt))

        pltpu.sync_copy(ks, osc_hbm.at[wid])
        pltpu.sync_copy(kx1, ox1_hbm.at[wid])
        pltpu.sync_copy(ky1, oy1_hbm.at[wid])
        pltpu.sync_copy(kx2, ox2_hbm.at[wid])
        pltpu.sync_copy(ky2, oy2_hbm.at[wid])
        pltpu.sync_copy(kn, on_hbm.at[wid])


@functools.partial(
    pl.kernel,
    out_type=[jax.ShapeDtypeStruct((OPAD * 5,), _f32),
              jax.ShapeDtypeStruct((OPAD,), _i32)],
    mesh=_mesh,
    compiler_params=pltpu.CompilerParams(needs_layout_passes=False, use_tc_tiling_on_sc=False),
    scratch_types=[
        pltpu.VMEM((C * KCAP + L,), _f32),
        pltpu.VMEM((C * KCAP + L,), _f32),
        pltpu.VMEM((C * KCAP + L,), _f32),
        pltpu.VMEM((C * KCAP + L,), _f32),
        pltpu.VMEM((C * KCAP + L,), _f32),
        pltpu.VMEM((C * KCAP + L,), _i32),
        pltpu.VMEM((2 * L,), _f32),
        pltpu.VMEM((2 * L,), _i32),
        pltpu.SMEM((2 * L,), _i32),
        pltpu.VMEM((OPAD * 5,), _f32),
        pltpu.VMEM((OPAD,), _i32),
    ],
)
def _merge_kernel(sc_hbm, x1_hbm, y1_hbm, x2_hbm, y2_hbm, n_hbm,
                  dets_hbm, labels_hbm,
                  vsc, vx1, vy1, vx2, vy2, vn, hs, hn, hp, dv, lv):
    wid = _wid()
    iota = lax.iota(_i32, L)
    lane0 = iota == 0

    @pl.when(wid == 0)
    def _():
        pltpu.sync_copy(sc_hbm, vsc.at[pl.ds(0, C * KCAP)])
        pltpu.sync_copy(x1_hbm, vx1.at[pl.ds(0, C * KCAP)])
        pltpu.sync_copy(y1_hbm, vy1.at[pl.ds(0, C * KCAP)])
        pltpu.sync_copy(x2_hbm, vx2.at[pl.ds(0, C * KCAP)])
        pltpu.sync_copy(y2_hbm, vy2.at[pl.ds(0, C * KCAP)])
        pltpu.sync_copy(n_hbm, vn.at[pl.ds(0, C * KCAP)])

        for half in range(2):
            cv = iota + half * L
            cidx = jnp.minimum(cv, C - 1) * KCAP
            h = plsc.load_gather(vsc, [cidx])
            hs[pl.ds(half * L, L)] = jnp.where(cv < C, h, _f32(-1.0))
            nh = plsc.load_gather(vn, [cidx])
            hn[pl.ds(half * L, L)] = jnp.where(cv < C, nh, _i32(0))

        def pinit(c, _):
            hp[c] = _i32(0)
            return 0

        lax.fori_loop(0, 2 * L, pinit, 0)

        def oinit(k, _):
            lv[pl.ds(k * L, L)] = jnp.full((L,), -1, _i32)
            return 0

        lax.fori_loop(0, OPAD // L, oinit, 0)

        def zinit(k, _):
            dv[pl.ds(k * L, L)] = jnp.zeros((L,), _f32)
            return 0

        lax.fori_loop(0, OPAD * 5 // L, zinit, 0)

        def mbody(k, _):
            h1 = hs[pl.ds(0, L)]
            h2 = hs[pl.ds(L, L)]
            m = jnp.maximum(jnp.max(h1), jnp.max(h2))

            @pl.when(m > _f32(0.0))
            def _():
                n1 = hn[pl.ds(0, L)]
                n2 = hn[pl.ds(L, L)]
                fi1 = jnp.where(h1 == m, n1 * C + iota, BIG)
                fi2 = jnp.where(h2 == m, n2 * C + (iota + L), BIG)
                fi = jnp.minimum(jnp.min(fi1), jnp.min(fi2))
                csel = lax.rem(fi, _i32(C))
                p = hp[csel]
                base = csel * KCAP + p
                _sstore(dv, k * 5 + 0, _sload(vx1, base), lane0)
                _sstore(dv, k * 5 + 1, _sload(vy1, base), lane0)
                _sstore(dv, k * 5 + 2, _sload(vx2, base), lane0)
                _sstore(dv, k * 5 + 3, _sload(vy2, base), lane0)
                _sstore(dv, k * 5 + 4, m, lane0)
                _sstore(lv, k, csel, lane0)
                pn = p + 1
                hp[csel] = pn
                pc = jnp.minimum(pn, KCAP - 1)
                nxt = _sload(vsc, csel * KCAP + pc)
                _sstore(hs, csel, jnp.where(pn > KCAP - 1, _f32(-1.0), nxt),
                        lane0)
                _sstore(hn, csel, _sload(vn, csel * KCAP + pc), lane0)

            return 0

        lax.fori_loop(0, MAX_NUM, mbody, 0)

        pltpu.sync_copy(dv, dets_hbm)
        pltpu.sync_copy(lv, labels_hbm)


def kernel(multi_bboxes, multi_scores):
    bb = multi_bboxes.reshape(N, C, 4)
    sc = multi_scores[:, :C]
    pad = ((0, 0), (0, NP - N))
    x1t = jnp.pad(bb[:, :, 0].T, pad)
    y1t = jnp.pad(bb[:, :, 1].T, pad)
    x2t = jnp.pad(bb[:, :, 2].T, pad)
    y2t = jnp.pad(bb[:, :, 3].T, pad)
    st = jnp.pad(sc.T, pad)
    maxv = _max_kernel(x2t, y2t)
    k_sc, k_x1, k_y1, k_x2, k_y2, k_n = _nms_kernel(
        st, x1t, y1t, x2t, y2t, maxv)
    dets_pad, labels_pad = _merge_kernel(
        k_sc.reshape(-1), k_x1.reshape(-1), k_y1.reshape(-1),
        k_x2.reshape(-1), k_y2.reshape(-1), k_n.reshape(-1))
    dets = dets_pad.reshape(OPAD, 5)[:MAX_NUM]
    labels = labels_pad[:MAX_NUM]
    return dets, labels


# speedup vs baseline: 19.9027x; 1.1397x over previous
import functools

import jax
import jax.numpy as jnp
import numpy as np
from jax import lax
from jax.experimental import pallas as pl
from jax.experimental.pallas import tpu as pltpu
from jax.experimental.pallas import tpu_sc as plsc

SCORE_THR = 0.05
MAX_NUM = 100
N = 5000
C = 20
L = 16
NP = 5008
NV = NP // L
KCAP = 128
OPAD = 128
NCORES = 2
NSUB = 16
BIG = np.int32(1 << 30)

_mesh = plsc.VectorSubcoreMesh(
    core_axis_name="c", subcore_axis_name="s",
    num_cores=NCORES, num_subcores=NSUB)

_f32 = np.float32
_i32 = np.int32


def _wid():
    return lax.axis_index("s") * NCORES + lax.axis_index("c")


def _sload(ref, idx):
    return ref[pl.ds(idx, L)][0]


def _sstore(ref, idx, val, lane0):
    plsc.store_scatter(
        ref, [jnp.full((L,), idx, _i32)], jnp.full((L,), val), mask=lane0)


@functools.partial(
    pl.kernel,
    out_type=jax.ShapeDtypeStruct((C * L,), _f32),
    mesh=_mesh,
    compiler_params=pltpu.CompilerParams(needs_layout_passes=False, use_tc_tiling_on_sc=False),
    scratch_types=[
        pltpu.VMEM((NP,), _f32),
        pltpu.VMEM((L,), _f32),
    ],
)
def _max_kernel(x2_hbm, y2_hbm, out_hbm, buf, mv):
    wid = _wid()

    @pl.when(wid < C)
    def _():
        def body(j, acc):
            return jnp.maximum(acc, buf[pl.ds(j * L, L)])

        pltpu.sync_copy(x2_hbm.at[wid], buf)
        acc = lax.fori_loop(0, NV, body, jnp.full((L,), -1e30, _f32))
        pltpu.sync_copy(y2_hbm.at[wid], buf)
        acc = lax.fori_loop(0, NV, body, acc)
        mv[...] = acc
        pltpu.sync_copy(mv, out_hbm.at[pl.ds(wid * L, L)])


NB = 256
CH = 112


@functools.partial(
    pl.kernel,
    out_type=[jax.ShapeDtypeStruct((C, KCAP), _f32)] * 5
    + [jax.ShapeDtypeStruct((C, KCAP), _i32)],
    mesh=_mesh,
    compiler_params=pltpu.CompilerParams(needs_layout_passes=False, use_tc_tiling_on_sc=False),
    scratch_types=[
        pltpu.VMEM((NP,), _f32),
        pltpu.VMEM((NP + L,), _f32),
        pltpu.VMEM((NP + L,), _f32),
        pltpu.VMEM((NP + L,), _f32),
        pltpu.VMEM((NP + L,), _f32),
        pltpu.VMEM((NP,), _i32),
        pltpu.VMEM((NB * L,), _i32),
        pltpu.VMEM((C * L,), _f32),
        pltpu.VMEM((NP + L,), _f32),
        pltpu.VMEM((NP + L,), _i32),
        pltpu.VMEM((NP + L,), _f32),
        pltpu.VMEM((NP + L,), _f32),
        pltpu.VMEM((NP + L,), _f32),
        pltpu.VMEM((NP + L,), _f32),
        pltpu.VMEM((NP + L,), _f32),
        pltpu.VMEM((KCAP,), _f32),
        pltpu.VMEM((KCAP,), _f32),
        pltpu.VMEM((KCAP,), _f32),
        pltpu.VMEM((KCAP,), _f32),
        pltpu.VMEM((KCAP,), _f32),
        pltpu.VMEM((KCAP,), _i32),
        pltpu.VMEM((KCAP,), _f32),
        pltpu.VMEM((KCAP,), _f32),
        pltpu.VMEM((KCAP,), _f32),
        pltpu.VMEM((KCAP,), _f32),
        pltpu.VMEM((KCAP,), _f32),
    ],
)
def _nms_kernel(s_hbm, x1_hbm, y1_hbm, x2_hbm, y2_hbm, maxv_hbm,
                osc_hbm, ox1_hbm, oy1_hbm, ox2_hbm, oy2_hbm, on_hbm,
                sv, gx1, gy1, gx2, gy2,
                bkt, hist, mbuf, cs, cidx, cx1, cy1, cx2, cy2, car,
                ks, kx1, ky1, kx2, ky2, kn, kbx1, kby1, kbx2, kby2, kbar):
    wid = _wid()
    iota = lax.iota(_i32, L)
    lane0 = iota == 0
    ones = jnp.ones((L,), _i32)

    @pl.when(wid < C)
    def _():
        pltpu.sync_copy(maxv_hbm, mbuf)
        pltpu.sync_copy(s_hbm.at[wid], sv)
        pltpu.sync_copy(x1_hbm.at[wid], gx1.at[pl.ds(0, NP)])
        pltpu.sync_copy(y1_hbm.at[wid], gy1.at[pl.ds(0, NP)])
        pltpu.sync_copy(x2_hbm.at[wid], gx2.at[pl.ds(0, NP)])
        pltpu.sync_copy(y2_hbm.at[wid], gy2.at[pl.ds(0, NP)])

        def mbody(j, acc):
            return jnp.maximum(acc, mbuf[pl.ds(j * L, L)])

        mxv = lax.fori_loop(0, C, mbody, jnp.full((L,), -1e30, _f32))
        gmax = jnp.max(mxv)
        off = wid.astype(_f32) * (gmax + _f32(1.0))

        def hz(j, _):
            hist[pl.ds(j * L, L)] = jnp.zeros((L,), _i32)
            return 0

        lax.fori_loop(0, NB, hz, 0)

        def p1(j, vcnt):
            sl = pl.ds(j * L, L)
            s = sv[sl]
            s = jnp.where(s > _f32(SCORE_THR), s, _f32(-1.0))
            sv[sl] = s
            valid = s > _f32(0.0)
            b = jnp.clip((s * _f32(NB)).astype(_i32), 0, NB - 1)
            bkt[sl] = jnp.where(valid, b, -1)
            plsc.addupdate_scatter(hist, [b * L + iota], ones, mask=valid)
            return vcnt + plsc.all_reduce_population_count(valid)[0]

        vcnt = lax.fori_loop(0, NV, p1, _i32(0))

        def ki(j, _):
            sl = pl.ds(j * L, L)
            ks[sl] = jnp.full((L,), -1.0, _f32)
            kx1[sl] = jnp.zeros((L,), _f32)
            ky1[sl] = jnp.zeros((L,), _f32)
            kx2[sl] = jnp.zeros((L,), _f32)
            ky2[sl] = jnp.zeros((L,), _f32)
            kn[sl] = jnp.zeros((L,), _i32)
            return 0

        lax.fori_loop(0, KCAP // L, ki, 0)

        def outer_cond(st):
            cnt, bp, rem = st
            return (cnt < MAX_NUM) & (rem > 0) & (bp >= 0)

        def outer_body(st):
            cnt, bp, rem = st

            def wcond(ws):
                acc, bptr = ws
                return (acc < CH) & (bptr >= 0)

            def wbody(ws):
                acc, bptr = ws
                cb = jnp.sum(hist[pl.ds(bptr * L, L)])
                return acc + cb, bptr - 1

            acc, bptr = lax.while_loop(wcond, wbody, (_i32(0), bp))
            b_lo = bptr + 1
            rem = rem - acc

            def coll(j, wp):
                sl = pl.ds(j * L, L)
                b = bkt[sl]
                msk = (b >= b_lo) & (b <= bp)
                plsc.store_compressed(cs.at[pl.ds(wp, L)], sv[sl], mask=msk)
                plsc.store_compressed(cidx.at[pl.ds(wp, L)],
                                      j * L + iota, mask=msk)
                return wp + plsc.all_reduce_population_count(msk)[0]

            m_sz = lax.fori_loop(0, NV, coll, _i32(0))
            cs[pl.ds(m_sz, L)] = jnp.full((L,), -1.0, _f32)
            cidx[pl.ds(m_sz, L)] = jnp.zeros((L,), _i32)
            mv = lax.div(m_sz + (L - 1), _i32(L))

            def cg(j, _):
                sl = pl.ds(j * L, L)
                iv = cidx[sl]
                x1 = plsc.load_gather(gx1, [iv]) + off
                y1 = plsc.load_gather(gy1, [iv]) + off
                x2 = plsc.load_gather(gx2, [iv]) + off
                y2 = plsc.load_gather(gy2, [iv]) + off
                cx1[sl] = x1
                cy1[sl] = y1
                cx2[sl] = x2
                cy2[sl] = y2
                car[sl] = (x2 - x1) * (y2 - y1)
                return 0

            lax.fori_loop(0, mv, cg, 0)

            def kchk(k, _):
                bx1 = _sload(kbx1, k)
                by1 = _sload(kby1, k)
                bx2 = _sload(kbx2, k)
                by2 = _sload(kby2, k)
                bar = _sload(kbar, k)

                def kchk_j(j, _2):
                    sl = pl.ds(j * L, L)
                    inter = (jnp.maximum(
                        jnp.minimum(bx2, cx2[sl]) - jnp.maximum(bx1, cx1[sl]),
                        _f32(0.0))
                        * jnp.maximum(
                        jnp.minimum(by2, cy2[sl]) - jnp.maximum(by1, cy1[sl]),
                        _f32(0.0)))
                    den = (bar + car[sl]) - inter + _f32(1e-9)
                    cs[sl] = jnp.where(_f32(2.0) * inter > den, _f32(-1.0),
                                       cs[sl])
                    return 0

                lax.fori_loop(0, mv, kchk_j, 0)
                return 0

            lax.fori_loop(0, cnt, kchk, 0)

            def am(j, carry):
                bv, bi = carry
                s = cs[pl.ds(j * L, L)]
                li = j * L + iota
                upd = s > bv
                return jnp.where(upd, s, bv), jnp.where(upd, li, bi)

            bv0, bi0 = lax.fori_loop(
                0, mv, am,
                (jnp.full((L,), -2.0, _f32), jnp.zeros((L,), _i32)))

            def cond(carry):
                cnt2, m, _, _ = carry
                return (cnt2 < MAX_NUM) & (m > _f32(0.0))

            def body(carry):
                cnt2, m, bv, bi = carry
                cand = jnp.where(bv == m, bi, BIG)
                lsel = jnp.full((L,), jnp.min(cand), _i32)
                nsel = plsc.load_gather(cidx, [lsel])
                cntv = jnp.full((L,), cnt2, _i32)
                mvz = jnp.full((L,), m, _f32)
                plsc.store_scatter(ks, [cntv], mvz, mask=lane0)
                plsc.store_scatter(kx1, [cntv],
                                   plsc.load_gather(gx1, [nsel]), mask=lane0)
                plsc.store_scatter(ky1, [cntv],
                                   plsc.load_gather(gy1, [nsel]), mask=lane0)
                plsc.store_scatter(kx2, [cntv],
                                   plsc.load_gather(gx2, [nsel]), mask=lane0)
                plsc.store_scatter(ky2, [cntv],
                                   plsc.load_gather(gy2, [nsel]), mask=lane0)
                plsc.store_scatter(kn, [cntv], nsel, mask=lane0)
                bx1 = plsc.load_gather(cx1, [lsel])
                by1 = plsc.load_gather(cy1, [lsel])
                bx2 = plsc.load_gather(cx2, [lsel])
                by2 = plsc.load_gather(cy2, [lsel])
                bar = plsc.load_gather(car, [lsel])
                plsc.store_scatter(kbx1, [cntv], bx1, mask=lane0)
                plsc.store_scatter(kby1, [cntv], by1, mask=lane0)
                plsc.store_scatter(kbx2, [cntv], bx2, mask=lane0)
                plsc.store_scatter(kby2, [cntv], by2, mask=lane0)
                plsc.store_scatter(kbar, [cntv], bar, mask=lane0)

                def sup(j, carry2):
                    bv2, bi2 = carry2
                    sl = pl.ds(j * L, L)
                    s = cs[sl]
                    inter = (jnp.maximum(
                        jnp.minimum(bx2, cx2[sl]) - jnp.maximum(bx1, cx1[sl]),
                        _f32(0.0))
                        * jnp.maximum(
                        jnp.minimum(by2, cy2[sl]) - jnp.maximum(by1, cy1[sl]),
                        _f32(0.0)))
                    den = (bar + car[sl]) - inter + _f32(1e-9)
                    s = jnp.where(_f32(2.0) * inter > den, _f32(-1.0), s)
                    cs[sl] = s
                    li = j * L + iota
                    upd = s > bv2
                    return (jnp.where(upd, s, bv2), jnp.where(upd, li, bi2))

                bv2, bi2 = lax.fori_loop(
                    0, mv, sup,
                    (jnp.full((L,), -2.0, _f32), jnp.zeros((L,), _i32)))
                return cnt2 + 1, jnp.max(bv2), bv2, bi2

            cnt, _, _, _ = lax.while_loop(
                cond, body, (cnt, jnp.max(bv0), bv0, bi0))
            return cnt, bptr, rem

        lax.while_loop(outer_cond, outer_body, (_i32(0), _i32(NB - 1), vcnt))

        pltpu.sync_copy(ks, osc_hbm.at[wid])
        pltpu.sync_copy(kx1, ox1_hbm.at[wid])
        pltpu.sync_copy(ky1, oy1_hbm.at[wid])
        pltpu.sync_copy(kx2, ox2_hbm.at[wid])
        pltpu.sync_copy(ky2, oy2_hbm.at[wid])
        pltpu.sync_copy(kn, on_hbm.at[wid])


@functools.partial(
    pl.kernel,
    out_type=[jax.ShapeDtypeStruct((OPAD * 5,), _f32),
              jax.ShapeDtypeStruct((OPAD,), _i32)],
    mesh=_mesh,
    compiler_params=pltpu.CompilerParams(needs_layout_passes=False, use_tc_tiling_on_sc=False),
    scratch_types=[
        pltpu.VMEM((C * KCAP + L,), _f32),
        pltpu.VMEM((C * KCAP + L,), _f32),
        pltpu.VMEM((C * KCAP + L,), _f32),
        pltpu.VMEM((C * KCAP + L,), _f32),
        pltpu.VMEM((C * KCAP + L,), _f32),
        pltpu.VMEM((C * KCAP + L,), _i32),
        pltpu.VMEM((2 * L,), _f32),
        pltpu.VMEM((2 * L,), _i32),
        pltpu.SMEM((2 * L,), _i32),
        pltpu.VMEM((OPAD * 5,), _f32),
        pltpu.VMEM((OPAD,), _i32),
    ],
)
def _merge_kernel(sc_hbm, x1_hbm, y1_hbm, x2_hbm, y2_hbm, n_hbm,
                  dets_hbm, labels_hbm,
                  vsc, vx1, vy1, vx2, vy2, vn, hs, hn, hp, dv, lv):
    wid = _wid()
    iota = lax.iota(_i32, L)
    lane0 = iota == 0

    @pl.when(wid == 0)
    def _():
        pltpu.sync_copy(sc_hbm, vsc.at[pl.ds(0, C * KCAP)])
        pltpu.sync_copy(x1_hbm, vx1.at[pl.ds(0, C * KCAP)])
        pltpu.sync_copy(y1_hbm, vy1.at[pl.ds(0, C * KCAP)])
        pltpu.sync_copy(x2_hbm, vx2.at[pl.ds(0, C * KCAP)])
        pltpu.sync_copy(y2_hbm, vy2.at[pl.ds(0, C * KCAP)])
        pltpu.sync_copy(n_hbm, vn.at[pl.ds(0, C * KCAP)])

        for half in range(2):
            cv = iota + half * L
            cidx = jnp.minimum(cv, C - 1) * KCAP
            h = plsc.load_gather(vsc, [cidx])
            hs[pl.ds(half * L, L)] = jnp.where(cv < C, h, _f32(-1.0))
            nh = plsc.load_gather(vn, [cidx])
            hn[pl.ds(half * L, L)] = jnp.where(cv < C, nh, _i32(0))

        def pinit(c, _):
            hp[c] = _i32(0)
            return 0

        lax.fori_loop(0, 2 * L, pinit, 0)

        def oinit(k, _):
            lv[pl.ds(k * L, L)] = jnp.full((L,), -1, _i32)
            return 0

        lax.fori_loop(0, OPAD // L, oinit, 0)

        def zinit(k, _):
            dv[pl.ds(k * L, L)] = jnp.zeros((L,), _f32)
            return 0

        lax.fori_loop(0, OPAD * 5 // L, zinit, 0)

        def mbody(k, _):
            h1 = hs[pl.ds(0, L)]
            h2 = hs[pl.ds(L, L)]
            m = jnp.maximum(jnp.max(h1), jnp.max(h2))

            @pl.when(m > _f32(0.0))
            def _():
                n1 = hn[pl.ds(0, L)]
                n2 = hn[pl.ds(L, L)]
                fi1 = jnp.where(h1 == m, n1 * C + iota, BIG)
                fi2 = jnp.where(h2 == m, n2 * C + (iota + L), BIG)
                fi = jnp.minimum(jnp.min(fi1), jnp.min(fi2))
                csel = lax.rem(fi, _i32(C))
                p = hp[csel]
                base = csel * KCAP + p
                _sstore(dv, k * 5 + 0, _sload(vx1, base), lane0)
                _sstore(dv, k * 5 + 1, _sload(vy1, base), lane0)
                _sstore(dv, k * 5 + 2, _sload(vx2, base), lane0)
                _sstore(dv, k * 5 + 3, _sload(vy2, base), lane0)
                _sstore(dv, k * 5 + 4, m, lane0)
                _sstore(lv, k, csel, lane0)
                pn = p + 1
                hp[csel] = pn
                pc = jnp.minimum(pn, KCAP - 1)
                nxt = _sload(vsc, csel * KCAP + pc)
                _sstore(hs, csel, jnp.where(pn > KCAP - 1, _f32(-1.0), nxt),
                        lane0)
                _sstore(hn, csel, _sload(vn, csel * KCAP + pc), lane0)

            return 0

        lax.fori_loop(0, MAX_NUM, mbody, 0)

        pltpu.sync_copy(dv, dets_hbm)
        pltpu.sync_copy(lv, labels_hbm)


def kernel(multi_bboxes, multi_scores):
    bb = multi_bboxes.reshape(N, C, 4)
    sc = multi_scores[:, :C]
    pad = ((0, 0), (0, NP - N))
    x1t = jnp.pad(bb[:, :, 0].T, pad)
    y1t = jnp.pad(bb[:, :, 1].T, pad)
    x2t = jnp.pad(bb[:, :, 2].T, pad)
    y2t = jnp.pad(bb[:, :, 3].T, pad)
    st = jnp.pad(sc.T, pad)
    maxv = _max_kernel(x2t, y2t)
    k_sc, k_x1, k_y1, k_x2, k_y2, k_n = _nms_kernel(
        st, x1t, y1t, x2t, y2t, maxv)
    dets_pad, labels_pad = _merge_kernel(
        k_sc.reshape(-1), k_x1.reshape(-1), k_y1.reshape(-1),
        k_x2.reshape(-1), k_y2.reshape(-1), k_n.reshape(-1))
    dets = dets_pad.reshape(OPAD, 5)[:MAX_NUM]
    labels = labels_pad[:MAX_NUM]
    return dets, labels


# speedup vs baseline: 20.1161x; 1.0107x over previous
import functools

import jax
import jax.numpy as jnp
import numpy as np
from jax import lax
from jax.experimental import pallas as pl
from jax.experimental.pallas import tpu as pltpu
from jax.experimental.pallas import tpu_sc as plsc

SCORE_THR = 0.05
MAX_NUM = 100
N = 5000
C = 20
L = 16
NP = 5008
NV = NP // L
KCAP = 128
OPAD = 128
NCORES = 2
NSUB = 16
BIG = np.int32(1 << 30)

_mesh = plsc.VectorSubcoreMesh(
    core_axis_name="c", subcore_axis_name="s",
    num_cores=NCORES, num_subcores=NSUB)

_f32 = np.float32
_i32 = np.int32


def _wid():
    return lax.axis_index("s") * NCORES + lax.axis_index("c")


def _sload(ref, idx):
    return ref[pl.ds(idx, L)][0]


def _sstore(ref, idx, val, lane0):
    plsc.store_scatter(
        ref, [jnp.full((L,), idx, _i32)], jnp.full((L,), val), mask=lane0)


NB = 256
CH = 112
E = C * NP
SLE = 6272
SLV = SLE // L


@functools.partial(
    pl.kernel,
    out_type=[jax.ShapeDtypeStruct((C, KCAP), _f32)] * 5
    + [jax.ShapeDtypeStruct((C, KCAP), _i32)],
    mesh=_mesh,
    compiler_params=pltpu.CompilerParams(needs_layout_passes=False, use_tc_tiling_on_sc=False),
    scratch_types=[
        pltpu.VMEM((NP,), _f32),
        pltpu.VMEM((NP + L,), _f32),
        pltpu.VMEM((NP + L,), _f32),
        pltpu.VMEM((NP + L,), _f32),
        pltpu.VMEM((NP + L,), _f32),
        pltpu.VMEM((NP,), _i32),
        pltpu.VMEM((NB * L,), _i32),
        pltpu.VMEM((NSUB * L,), _f32),
        pltpu.VMEM((SLE,), _f32),
        pltpu.VMEM((L,), _f32),
        pltpu.VMEM_SHARED((NSUB * L,), _f32),
        pltpu.VMEM((NP + L,), _f32),
        pltpu.VMEM((NP + L,), _i32),
        pltpu.VMEM((NP + L,), _f32),
        pltpu.VMEM((NP + L,), _f32),
        pltpu.VMEM((NP + L,), _f32),
        pltpu.VMEM((NP + L,), _f32),
        pltpu.VMEM((NP + L,), _f32),
        pltpu.VMEM((KCAP,), _f32),
        pltpu.VMEM((KCAP,), _f32),
        pltpu.VMEM((KCAP,), _f32),
        pltpu.VMEM((KCAP,), _f32),
        pltpu.VMEM((KCAP,), _f32),
        pltpu.VMEM((KCAP,), _i32),
        pltpu.VMEM((KCAP,), _f32),
        pltpu.VMEM((KCAP,), _f32),
        pltpu.VMEM((KCAP,), _f32),
        pltpu.VMEM((KCAP,), _f32),
        pltpu.VMEM((KCAP,), _f32),
    ],
)
def _nms_kernel(s_hbm, x1_hbm, y1_hbm, x2_hbm, y2_hbm,
                osc_hbm, ox1_hbm, oy1_hbm, ox2_hbm, oy2_hbm, on_hbm,
                sv, gx1, gy1, gx2, gy2,
                bkt, hist, mbuf, slab, accb, shm, cs, cidx,
                cx1, cy1, cx2, cy2, car,
                ks, kx1, ky1, kx2, ky2, kn, kbx1, kby1, kbx2, kby2, kbar):
    wid = _wid()
    iota = lax.iota(_i32, L)
    lane0 = iota == 0
    ones = jnp.ones((L,), _i32)

    sid = lax.axis_index("s")
    start = jnp.minimum(sid * SLE, E - SLE)

    def mslab(j, acc):
        return jnp.maximum(acc, slab[pl.ds(j * L, L)])

    pltpu.sync_copy(x2_hbm.at[pl.ds(start, SLE)], slab)
    acc = lax.fori_loop(0, SLV, mslab, jnp.full((L,), -1e30, _f32))
    pltpu.sync_copy(y2_hbm.at[pl.ds(start, SLE)], slab)
    acc = lax.fori_loop(0, SLV, mslab, acc)
    accb[...] = acc
    pltpu.sync_copy(accb, shm.at[pl.ds(sid * L, L)])
    plsc.subcore_barrier()
    pltpu.sync_copy(shm, mbuf)

    def mbody(j, acc):
        return jnp.maximum(acc, mbuf[pl.ds(j * L, L)])

    mxv = lax.fori_loop(0, NSUB, mbody, jnp.full((L,), -1e30, _f32))
    gmax = jnp.max(mxv)

    @pl.when(wid < C)
    def _():
        row = pl.ds(wid * NP, NP)
        pltpu.sync_copy(s_hbm.at[row], sv)
        pltpu.sync_copy(x1_hbm.at[row], gx1.at[pl.ds(0, NP)])
        pltpu.sync_copy(y1_hbm.at[row], gy1.at[pl.ds(0, NP)])
        pltpu.sync_copy(x2_hbm.at[row], gx2.at[pl.ds(0, NP)])
        pltpu.sync_copy(y2_hbm.at[row], gy2.at[pl.ds(0, NP)])
        off = wid.astype(_f32) * (gmax + _f32(1.0))

        def hz(j, _):
            hist[pl.ds(j * L, L)] = jnp.zeros((L,), _i32)
            return 0

        lax.fori_loop(0, NB, hz, 0)

        def p1(j, vcnt):
            sl = pl.ds(j * L, L)
            s = sv[sl]
            s = jnp.where(s > _f32(SCORE_THR), s, _f32(-1.0))
            sv[sl] = s
            valid = s > _f32(0.0)
            b = jnp.clip((s * _f32(NB)).astype(_i32), 0, NB - 1)
            bkt[sl] = jnp.where(valid, b, -1)
            plsc.addupdate_scatter(hist, [b * L + iota], ones, mask=valid)
            return vcnt + plsc.all_reduce_population_count(valid)[0]

        vcnt = lax.fori_loop(0, NV, p1, _i32(0))

        def ki(j, _):
            sl = pl.ds(j * L, L)
            ks[sl] = jnp.full((L,), -1.0, _f32)
            kx1[sl] = jnp.zeros((L,), _f32)
            ky1[sl] = jnp.zeros((L,), _f32)
            kx2[sl] = jnp.zeros((L,), _f32)
            ky2[sl] = jnp.zeros((L,), _f32)
            kn[sl] = jnp.zeros((L,), _i32)
            return 0

        lax.fori_loop(0, KCAP // L, ki, 0)

        def outer_cond(st):
            cnt, bp, rem = st
            return (cnt < MAX_NUM) & (rem > 0) & (bp >= 0)

        def outer_body(st):
            cnt, bp, rem = st

            def wcond(ws):
                acc, bptr = ws
                return (acc < CH) & (bptr >= 0)

            def wbody(ws):
                acc, bptr = ws
                cb = jnp.sum(hist[pl.ds(bptr * L, L)])
                return acc + cb, bptr - 1

            acc, bptr = lax.while_loop(wcond, wbody, (_i32(0), bp))
            b_lo = bptr + 1
            rem = rem - acc

            def coll(j, wp):
                sl = pl.ds(j * L, L)
                b = bkt[sl]
                msk = (b >= b_lo) & (b <= bp)
                plsc.store_compressed(cs.at[pl.ds(wp, L)], sv[sl], mask=msk)
                plsc.store_compressed(cidx.at[pl.ds(wp, L)],
                                      j * L + iota, mask=msk)
                return wp + plsc.all_reduce_population_count(msk)[0]

            m_sz = lax.fori_loop(0, NV, coll, _i32(0))
            cs[pl.ds(m_sz, L)] = jnp.full((L,), -1.0, _f32)
            cidx[pl.ds(m_sz, L)] = jnp.zeros((L,), _i32)
            mv = lax.div(m_sz + (L - 1), _i32(L))

            def cg(j, _):
                sl = pl.ds(j * L, L)
                iv = cidx[sl]
                x1 = plsc.load_gather(gx1, [iv]) + off
                y1 = plsc.load_gather(gy1, [iv]) + off
                x2 = plsc.load_gather(gx2, [iv]) + off
                y2 = plsc.load_gather(gy2, [iv]) + off
                cx1[sl] = x1
                cy1[sl] = y1
                cx2[sl] = x2
                cy2[sl] = y2
                car[sl] = (x2 - x1) * (y2 - y1)
                return 0

            lax.fori_loop(0, mv, cg, 0)

            def kchk(k, _):
                bx1 = _sload(kbx1, k)
                by1 = _sload(kby1, k)
                bx2 = _sload(kbx2, k)
                by2 = _sload(kby2, k)
                bar = _sload(kbar, k)

                def kchk_j(j, _2):
                    sl = pl.ds(j * L, L)
                    inter = (jnp.maximum(
                        jnp.minimum(bx2, cx2[sl]) - jnp.maximum(bx1, cx1[sl]),
                        _f32(0.0))
                        * jnp.maximum(
                        jnp.minimum(by2, cy2[sl]) - jnp.maximum(by1, cy1[sl]),
                        _f32(0.0)))
                    den = (bar + car[sl]) - inter + _f32(1e-9)
                    cs[sl] = jnp.where(_f32(2.0) * inter > den, _f32(-1.0),
                                       cs[sl])
                    return 0

                lax.fori_loop(0, mv, kchk_j, 0)
                return 0

            lax.fori_loop(0, cnt, kchk, 0)

            def am(j, carry):
                bv, bi = carry
                s = cs[pl.ds(j * L, L)]
                li = j * L + iota
                upd = s > bv
                return jnp.where(upd, s, bv), jnp.where(upd, li, bi)

            bv0, bi0 = lax.fori_loop(
                0, mv, am,
                (jnp.full((L,), -2.0, _f32), jnp.zeros((L,), _i32)))

            def cond(carry):
                cnt2, m, _, _ = carry
                return (cnt2 < MAX_NUM) & (m > _f32(0.0))

            def body(carry):
                cnt2, m, bv, bi = carry
                cand = jnp.where(bv == m, bi, BIG)
                lsel = jnp.full((L,), jnp.min(cand), _i32)
                nsel = plsc.load_gather(cidx, [lsel])
                cntv = jnp.full((L,), cnt2, _i32)
                mvz = jnp.full((L,), m, _f32)
                plsc.store_scatter(ks, [cntv], mvz, mask=lane0)
                plsc.store_scatter(kx1, [cntv],
                                   plsc.load_gather(gx1, [nsel]), mask=lane0)
                plsc.store_scatter(ky1, [cntv],
                                   plsc.load_gather(gy1, [nsel]), mask=lane0)
                plsc.store_scatter(kx2, [cntv],
                                   plsc.load_gather(gx2, [nsel]), mask=lane0)
                plsc.store_scatter(ky2, [cntv],
                                   plsc.load_gather(gy2, [nsel]), mask=lane0)
                plsc.store_scatter(kn, [cntv], nsel, mask=lane0)
                bx1 = plsc.load_gather(cx1, [lsel])
                by1 = plsc.load_gather(cy1, [lsel])
                bx2 = plsc.load_gather(cx2, [lsel])
                by2 = plsc.load_gather(cy2, [lsel])
                bar = plsc.load_gather(car, [lsel])
                plsc.store_scatter(kbx1, [cntv], bx1, mask=lane0)
                plsc.store_scatter(kby1, [cntv], by1, mask=lane0)
                plsc.store_scatter(kbx2, [cntv], bx2, mask=lane0)
                plsc.store_scatter(kby2, [cntv], by2, mask=lane0)
                plsc.store_scatter(kbar, [cntv], bar, mask=lane0)

                def sup(j, carry2):
                    bv2, bi2 = carry2
                    sl = pl.ds(j * L, L)
                    s = cs[sl]
                    inter = (jnp.maximum(
                        jnp.minimum(bx2, cx2[sl]) - jnp.maximum(bx1, cx1[sl]),
                        _f32(0.0))
                        * jnp.maximum(
                        jnp.minimum(by2, cy2[sl]) - jnp.maximum(by1, cy1[sl]),
                        _f32(0.0)))
                    den = (bar + car[sl]) - inter + _f32(1e-9)
                    s = jnp.where(_f32(2.0) * inter > den, _f32(-1.0), s)
                    cs[sl] = s
                    li = j * L + iota
                    upd = s > bv2
                    return (jnp.where(upd, s, bv2), jnp.where(upd, li, bi2))

                bv2, bi2 = lax.fori_loop(
                    0, mv, sup,
                    (jnp.full((L,), -2.0, _f32), jnp.zeros((L,), _i32)))
                return cnt2 + 1, jnp.max(bv2), bv2, bi2

            cnt, _, _, _ = lax.while_loop(
                cond, body, (cnt, jnp.max(bv0), bv0, bi0))
            return cnt, bptr, rem

        lax.while_loop(outer_cond, outer_body, (_i32(0), _i32(NB - 1), vcnt))

        pltpu.sync_copy(ks, osc_hbm.at[wid])
        pltpu.sync_copy(kx1, ox1_hbm.at[wid])
        pltpu.sync_copy(ky1, oy1_hbm.at[wid])
        pltpu.sync_copy(kx2, ox2_hbm.at[wid])
        pltpu.sync_copy(ky2, oy2_hbm.at[wid])
        pltpu.sync_copy(kn, on_hbm.at[wid])


@functools.partial(
    pl.kernel,
    out_type=[jax.ShapeDtypeStruct((OPAD * 5,), _f32),
              jax.ShapeDtypeStruct((OPAD,), _i32)],
    mesh=_mesh,
    compiler_params=pltpu.CompilerParams(needs_layout_passes=False, use_tc_tiling_on_sc=False),
    scratch_types=[
        pltpu.VMEM((C * KCAP + L,), _f32),
        pltpu.VMEM((C * KCAP + L,), _f32),
        pltpu.VMEM((C * KCAP + L,), _f32),
        pltpu.VMEM((C * KCAP + L,), _f32),
        pltpu.VMEM((C * KCAP + L,), _f32),
        pltpu.VMEM((C * KCAP + L,), _i32),
        pltpu.VMEM((2 * L,), _f32),
        pltpu.VMEM((2 * L,), _i32),
        pltpu.SMEM((2 * L,), _i32),
        pltpu.VMEM((OPAD * 5,), _f32),
        pltpu.VMEM((OPAD,), _i32),
    ],
)
def _merge_kernel(sc_hbm, x1_hbm, y1_hbm, x2_hbm, y2_hbm, n_hbm,
                  dets_hbm, labels_hbm,
                  vsc, vx1, vy1, vx2, vy2, vn, hs, hn, hp, dv, lv):
    wid = _wid()
    iota = lax.iota(_i32, L)
    lane0 = iota == 0

    @pl.when(wid == 0)
    def _():
        pltpu.sync_copy(sc_hbm, vsc.at[pl.ds(0, C * KCAP)])
        pltpu.sync_copy(x1_hbm, vx1.at[pl.ds(0, C * KCAP)])
        pltpu.sync_copy(y1_hbm, vy1.at[pl.ds(0, C * KCAP)])
        pltpu.sync_copy(x2_hbm, vx2.at[pl.ds(0, C * KCAP)])
        pltpu.sync_copy(y2_hbm, vy2.at[pl.ds(0, C * KCAP)])
        pltpu.sync_copy(n_hbm, vn.at[pl.ds(0, C * KCAP)])

        for half in range(2):
            cv = iota + half * L
            cidx = jnp.minimum(cv, C - 1) * KCAP
            h = plsc.load_gather(vsc, [cidx])
            hs[pl.ds(half * L, L)] = jnp.where(cv < C, h, _f32(-1.0))
            nh = plsc.load_gather(vn, [cidx])
            hn[pl.ds(half * L, L)] = jnp.where(cv < C, nh, _i32(0))

        def pinit(c, _):
            hp[c] = _i32(0)
            return 0

        lax.fori_loop(0, 2 * L, pinit, 0)

        def oinit(k, _):
            lv[pl.ds(k * L, L)] = jnp.full((L,), -1, _i32)
            return 0

        lax.fori_loop(0, OPAD // L, oinit, 0)

        def zinit(k, _):
            dv[pl.ds(k * L, L)] = jnp.zeros((L,), _f32)
            return 0

        lax.fori_loop(0, OPAD * 5 // L, zinit, 0)

        def mbody(k, _):
            h1 = hs[pl.ds(0, L)]
            h2 = hs[pl.ds(L, L)]
            m = jnp.maximum(jnp.max(h1), jnp.max(h2))

            @pl.when(m > _f32(0.0))
            def _():
                n1 = hn[pl.ds(0, L)]
                n2 = hn[pl.ds(L, L)]
                fi1 = jnp.where(h1 == m, n1 * C + iota, BIG)
                fi2 = jnp.where(h2 == m, n2 * C + (iota + L), BIG)
                fi = jnp.minimum(jnp.min(fi1), jnp.min(fi2))
                csel = lax.rem(fi, _i32(C))
                p = hp[csel]
                base = csel * KCAP + p
                _sstore(dv, k * 5 + 0, _sload(vx1, base), lane0)
                _sstore(dv, k * 5 + 1, _sload(vy1, base), lane0)
                _sstore(dv, k * 5 + 2, _sload(vx2, base), lane0)
                _sstore(dv, k * 5 + 3, _sload(vy2, base), lane0)
                _sstore(dv, k * 5 + 4, m, lane0)
                _sstore(lv, k, csel, lane0)
                pn = p + 1
                hp[csel] = pn
                pc = jnp.minimum(pn, KCAP - 1)
                nxt = _sload(vsc, csel * KCAP + pc)
                _sstore(hs, csel, jnp.where(pn > KCAP - 1, _f32(-1.0), nxt),
                        lane0)
                _sstore(hn, csel, _sload(vn, csel * KCAP + pc), lane0)

            return 0

        lax.fori_loop(0, MAX_NUM, mbody, 0)

        pltpu.sync_copy(dv, dets_hbm)
        pltpu.sync_copy(lv, labels_hbm)


def kernel(multi_bboxes, multi_scores):
    bb = multi_bboxes.reshape(N, C, 4)
    sc = multi_scores[:, :C]
    pad = ((0, 0), (0, NP - N))
    x1t = jnp.pad(bb[:, :, 0].T, pad)
    y1t = jnp.pad(bb[:, :, 1].T, pad)
    x2t = jnp.pad(bb[:, :, 2].T, pad)
    y2t = jnp.pad(bb[:, :, 3].T, pad)
    st = jnp.pad(sc.T, pad)
    k_sc, k_x1, k_y1, k_x2, k_y2, k_n = _nms_kernel(
        st.reshape(-1), x1t.reshape(-1), y1t.reshape(-1),
        x2t.reshape(-1), y2t.reshape(-1))
    dets_pad, labels_pad = _merge_kernel(
        k_sc.reshape(-1), k_x1.reshape(-1), k_y1.reshape(-1),
        k_x2.reshape(-1), k_y2.reshape(-1), k_n.reshape(-1))
    dets = dets_pad.reshape(OPAD, 5)[:MAX_NUM]
    labels = labels_pad[:MAX_NUM]
    return dets, labels


# speedup vs baseline: 20.6383x; 1.0260x over previous
import functools

import jax
import jax.numpy as jnp
import numpy as np
from jax import lax
from jax.experimental import pallas as pl
from jax.experimental.pallas import tpu as pltpu
from jax.experimental.pallas import tpu_sc as plsc

SCORE_THR = 0.05
MAX_NUM = 100
N = 5000
C = 20
L = 16
NP = 5008
NV = NP // L
KCAP = 128
OPAD = 128
NCORES = 2
NSUB = 16
BIG = np.int32(1 << 30)

_mesh = plsc.VectorSubcoreMesh(
    core_axis_name="c", subcore_axis_name="s",
    num_cores=NCORES, num_subcores=NSUB)

_f32 = np.float32
_i32 = np.int32


def _wid():
    return lax.axis_index("s") * NCORES + lax.axis_index("c")


def _sload(ref, idx):
    return ref[pl.ds(idx, L)][0]


def _sstore(ref, idx, val, lane0):
    plsc.store_scatter(
        ref, [jnp.full((L,), idx, _i32)], jnp.full((L,), val), mask=lane0)


NB = 256
CH = 112
NTASK = 2 * C


@functools.partial(
    pl.kernel,
    out_type=[jax.ShapeDtypeStruct((C, KCAP), _f32)] * 5
    + [jax.ShapeDtypeStruct((C, KCAP), _i32)],
    mesh=_mesh,
    compiler_params=pltpu.CompilerParams(needs_layout_passes=False, use_tc_tiling_on_sc=False),
    scratch_types=[
        pltpu.VMEM((NP,), _f32),
        pltpu.VMEM((NP + L,), _f32),
        pltpu.VMEM((NP + L,), _f32),
        pltpu.VMEM((NP + L,), _f32),
        pltpu.VMEM((NP + L,), _f32),
        pltpu.VMEM((NP,), _i32),
        pltpu.VMEM((NB * L,), _i32),
        pltpu.VMEM((NSUB * L,), _f32),
        pltpu.VMEM((NP,), _f32),
        pltpu.VMEM((L,), _f32),
        pltpu.VMEM_SHARED((NSUB * L,), _f32),
        pltpu.VMEM((NP + L,), _f32),
        pltpu.VMEM((NP + L,), _i32),
        pltpu.VMEM((NP + L,), _f32),
        pltpu.VMEM((NP + L,), _f32),
        pltpu.VMEM((NP + L,), _f32),
        pltpu.VMEM((NP + L,), _f32),
        pltpu.VMEM((NP + L,), _f32),
        pltpu.VMEM((KCAP,), _f32),
        pltpu.VMEM((KCAP,), _f32),
        pltpu.VMEM((KCAP,), _f32),
        pltpu.VMEM((KCAP,), _f32),
        pltpu.VMEM((KCAP,), _f32),
        pltpu.VMEM((KCAP,), _i32),
        pltpu.VMEM((KCAP,), _f32),
        pltpu.VMEM((KCAP,), _f32),
        pltpu.VMEM((KCAP,), _f32),
        pltpu.VMEM((KCAP,), _f32),
        pltpu.VMEM((KCAP,), _f32),
    ],
)
def _nms_kernel(s_hbm, bb_hbm,
                osc_hbm, ox1_hbm, oy1_hbm, ox2_hbm, oy2_hbm, on_hbm,
                sv, gx1, gy1, gx2, gy2,
                bkt, hist, mbuf, slab, accb, shm, cs, cidx,
                cx1, cy1, cx2, cy2, car,
                ks, kx1, ky1, kx2, ky2, kn, kbx1, kby1, kbx2, kby2, kbar):
    wid = _wid()
    iota = lax.iota(_i32, L)
    lane0 = iota == 0
    ones = jnp.ones((L,), _i32)

    sid = lax.axis_index("s")

    def mslab(j, acc):
        return jnp.maximum(acc, slab[pl.ds(j * L, L)])

    def mtask(t, acc):
        tt = jnp.minimum(sid + t * NSUB, NTASK - 1)
        r = 4 * lax.div(tt, 2) + 2 + lax.rem(tt, 2)
        pltpu.sync_copy(bb_hbm.at[pl.ds(r * NP, NP)], slab)
        return lax.fori_loop(0, NV, mslab, acc)

    acc = lax.fori_loop(0, (NTASK + NSUB - 1) // NSUB, mtask,
                        jnp.full((L,), -1e30, _f32))
    accb[...] = acc
    pltpu.sync_copy(accb, shm.at[pl.ds(sid * L, L)])
    plsc.subcore_barrier()
    pltpu.sync_copy(shm, mbuf)

    def mbody(j, acc):
        return jnp.maximum(acc, mbuf[pl.ds(j * L, L)])

    mxv = lax.fori_loop(0, NSUB, mbody, jnp.full((L,), -1e30, _f32))
    gmax = jnp.max(mxv)

    @pl.when(wid < C)
    def _():
        pltpu.sync_copy(s_hbm.at[pl.ds(wid * NP, NP)], sv)
        base = 4 * wid * NP
        pltpu.sync_copy(bb_hbm.at[pl.ds(base, NP)], gx1.at[pl.ds(0, NP)])
        pltpu.sync_copy(bb_hbm.at[pl.ds(base + NP, NP)],
                        gy1.at[pl.ds(0, NP)])
        pltpu.sync_copy(bb_hbm.at[pl.ds(base + 2 * NP, NP)],
                        gx2.at[pl.ds(0, NP)])
        pltpu.sync_copy(bb_hbm.at[pl.ds(base + 3 * NP, NP)],
                        gy2.at[pl.ds(0, NP)])
        off = wid.astype(_f32) * (gmax + _f32(1.0))

        def hz(j, _):
            hist[pl.ds(j * L, L)] = jnp.zeros((L,), _i32)
            return 0

        lax.fori_loop(0, NB, hz, 0)

        def p1(j, vcnt):
            sl = pl.ds(j * L, L)
            s = sv[sl]
            s = jnp.where(s > _f32(SCORE_THR), s, _f32(-1.0))
            sv[sl] = s
            valid = s > _f32(0.0)
            b = jnp.clip((s * _f32(NB)).astype(_i32), 0, NB - 1)
            bkt[sl] = jnp.where(valid, b, -1)
            plsc.addupdate_scatter(hist, [b * L + iota], ones, mask=valid)
            return vcnt + plsc.all_reduce_population_count(valid)[0]

        vcnt = lax.fori_loop(0, NV, p1, _i32(0))

        def ki(j, _):
            sl = pl.ds(j * L, L)
            ks[sl] = jnp.full((L,), -1.0, _f32)
            kx1[sl] = jnp.zeros((L,), _f32)
            ky1[sl] = jnp.zeros((L,), _f32)
            kx2[sl] = jnp.zeros((L,), _f32)
            ky2[sl] = jnp.zeros((L,), _f32)
            kn[sl] = jnp.zeros((L,), _i32)
            return 0

        lax.fori_loop(0, KCAP // L, ki, 0)

        def outer_cond(st):
            cnt, bp, rem = st
            return (cnt < MAX_NUM) & (rem > 0) & (bp >= 0)

        def outer_body(st):
            cnt, bp, rem = st

            def wcond(ws):
                acc, bptr = ws
                return (acc < CH) & (bptr >= 0)

            def wbody(ws):
                acc, bptr = ws
                cb = jnp.sum(hist[pl.ds(bptr * L, L)])
                return acc + cb, bptr - 1

            acc, bptr = lax.while_loop(wcond, wbody, (_i32(0), bp))
            b_lo = bptr + 1
            rem = rem - acc

            def coll(j, wp):
                sl = pl.ds(j * L, L)
                b = bkt[sl]
                msk = (b >= b_lo) & (b <= bp)
                plsc.store_compressed(cs.at[pl.ds(wp, L)], sv[sl], mask=msk)
                plsc.store_compressed(cidx.at[pl.ds(wp, L)],
                                      j * L + iota, mask=msk)
                return wp + plsc.all_reduce_population_count(msk)[0]

            m_sz = lax.fori_loop(0, NV, coll, _i32(0))
            cs[pl.ds(m_sz, L)] = jnp.full((L,), -1.0, _f32)
            cidx[pl.ds(m_sz, L)] = jnp.zeros((L,), _i32)
            mv = lax.div(m_sz + (L - 1), _i32(L))

            def cg(j, _):
                sl = pl.ds(j * L, L)
                iv = cidx[sl]
                x1 = plsc.load_gather(gx1, [iv]) + off
                y1 = plsc.load_gather(gy1, [iv]) + off
                x2 = plsc.load_gather(gx2, [iv]) + off
                y2 = plsc.load_gather(gy2, [iv]) + off
                cx1[sl] = x1
                cy1[sl] = y1
                cx2[sl] = x2
                cy2[sl] = y2
                car[sl] = (x2 - x1) * (y2 - y1)
                return 0

            lax.fori_loop(0, mv, cg, 0)

            def kchk(k, _):
                bx1 = _sload(kbx1, k)
                by1 = _sload(kby1, k)
                bx2 = _sload(kbx2, k)
                by2 = _sload(kby2, k)
                bar = _sload(kbar, k)

                def kchk_j(j, _2):
                    sl = pl.ds(j * L, L)
                    inter = (jnp.maximum(
                        jnp.minimum(bx2, cx2[sl]) - jnp.maximum(bx1, cx1[sl]),
                        _f32(0.0))
                        * jnp.maximum(
                        jnp.minimum(by2, cy2[sl]) - jnp.maximum(by1, cy1[sl]),
                        _f32(0.0)))
                    den = (bar + car[sl]) - inter + _f32(1e-9)
                    cs[sl] = jnp.where(_f32(2.0) * inter > den, _f32(-1.0),
                                       cs[sl])
                    return 0

                lax.fori_loop(0, mv, kchk_j, 0)
                return 0

            lax.fori_loop(0, cnt, kchk, 0)

            def am(j, carry):
                bv, bi = carry
                s = cs[pl.ds(j * L, L)]
                li = j * L + iota
                upd = s > bv
                return jnp.where(upd, s, bv), jnp.where(upd, li, bi)

            bv0, bi0 = lax.fori_loop(
                0, mv, am,
                (jnp.full((L,), -2.0, _f32), jnp.zeros((L,), _i32)))

            def cond(carry):
                cnt2, m, _, _ = carry
                return (cnt2 < MAX_NUM) & (m > _f32(0.0))

            def body(carry):
                cnt2, m, bv, bi = carry
                cand = jnp.where(bv == m, bi, BIG)
                lsel = jnp.full((L,), jnp.min(cand), _i32)
                nsel = plsc.load_gather(cidx, [lsel])
                cntv = jnp.full((L,), cnt2, _i32)
                mvz = jnp.full((L,), m, _f32)
                plsc.store_scatter(ks, [cntv], mvz, mask=lane0)
                plsc.store_scatter(kx1, [cntv],
                                   plsc.load_gather(gx1, [nsel]), mask=lane0)
                plsc.store_scatter(ky1, [cntv],
                                   plsc.load_gather(gy1, [nsel]), mask=lane0)
                plsc.store_scatter(kx2, [cntv],
                                   plsc.load_gather(gx2, [nsel]), mask=lane0)
                plsc.store_scatter(ky2, [cntv],
                                   plsc.load_gather(gy2, [nsel]), mask=lane0)
                plsc.store_scatter(kn, [cntv], nsel, mask=lane0)
                bx1 = plsc.load_gather(cx1, [lsel])
                by1 = plsc.load_gather(cy1, [lsel])
                bx2 = plsc.load_gather(cx2, [lsel])
                by2 = plsc.load_gather(cy2, [lsel])
                bar = plsc.load_gather(car, [lsel])
                plsc.store_scatter(kbx1, [cntv], bx1, mask=lane0)
                plsc.store_scatter(kby1, [cntv], by1, mask=lane0)
                plsc.store_scatter(kbx2, [cntv], bx2, mask=lane0)
                plsc.store_scatter(kby2, [cntv], by2, mask=lane0)
                plsc.store_scatter(kbar, [cntv], bar, mask=lane0)

                def sup(j, carry2):
                    bv2, bi2 = carry2
                    sl = pl.ds(j * L, L)
                    s = cs[sl]
                    inter = (jnp.maximum(
                        jnp.minimum(bx2, cx2[sl]) - jnp.maximum(bx1, cx1[sl]),
                        _f32(0.0))
                        * jnp.maximum(
                        jnp.minimum(by2, cy2[sl]) - jnp.maximum(by1, cy1[sl]),
                        _f32(0.0)))
                    den = (bar + car[sl]) - inter + _f32(1e-9)
                    s = jnp.where(_f32(2.0) * inter > den, _f32(-1.0), s)
                    cs[sl] = s
                    li = j * L + iota
                    upd = s > bv2
                    return (jnp.where(upd, s, bv2), jnp.where(upd, li, bi2))

                bv2, bi2 = lax.fori_loop(
                    0, mv, sup,
                    (jnp.full((L,), -2.0, _f32), jnp.zeros((L,), _i32)))
                return cnt2 + 1, jnp.max(bv2), bv2, bi2

            cnt, _, _, _ = lax.while_loop(
                cond, body, (cnt, jnp.max(bv0), bv0, bi0))
            return cnt, bptr, rem

        lax.while_loop(outer_cond, outer_body, (_i32(0), _i32(NB - 1), vcnt))

        pltpu.sync_copy(ks, osc_hbm.at[wid])
        pltpu.sync_copy(kx1, ox1_hbm.at[wid])
        pltpu.sync_copy(ky1, oy1_hbm.at[wid])
        pltpu.sync_copy(kx2, ox2_hbm.at[wid])
        pltpu.sync_copy(ky2, oy2_hbm.at[wid])
        pltpu.sync_copy(kn, on_hbm.at[wid])


@functools.partial(
    pl.kernel,
    out_type=[jax.ShapeDtypeStruct((OPAD * 5,), _f32),
              jax.ShapeDtypeStruct((OPAD,), _i32)],
    mesh=_mesh,
    compiler_params=pltpu.CompilerParams(needs_layout_passes=False, use_tc_tiling_on_sc=False),
    scratch_types=[
        pltpu.VMEM((C * KCAP + L,), _f32),
        pltpu.VMEM((C * KCAP + L,), _f32),
        pltpu.VMEM((C * KCAP + L,), _f32),
        pltpu.VMEM((C * KCAP + L,), _f32),
        pltpu.VMEM((C * KCAP + L,), _f32),
        pltpu.VMEM((C * KCAP + L,), _i32),
        pltpu.VMEM((2 * L,), _f32),
        pltpu.VMEM((2 * L,), _i32),
        pltpu.SMEM((2 * L,), _i32),
        pltpu.VMEM((OPAD * 5,), _f32),
        pltpu.VMEM((OPAD,), _i32),
    ],
)
def _merge_kernel(sc_hbm, x1_hbm, y1_hbm, x2_hbm, y2_hbm, n_hbm,
                  dets_hbm, labels_hbm,
                  vsc, vx1, vy1, vx2, vy2, vn, hs, hn, hp, dv, lv):
    wid = _wid()
    iota = lax.iota(_i32, L)
    lane0 = iota == 0

    @pl.when(wid == 0)
    def _():
        pltpu.sync_copy(sc_hbm, vsc.at[pl.ds(0, C * KCAP)])
        pltpu.sync_copy(x1_hbm, vx1.at[pl.ds(0, C * KCAP)])
        pltpu.sync_copy(y1_hbm, vy1.at[pl.ds(0, C * KCAP)])
        pltpu.sync_copy(x2_hbm, vx2.at[pl.ds(0, C * KCAP)])
        pltpu.sync_copy(y2_hbm, vy2.at[pl.ds(0, C * KCAP)])
        pltpu.sync_copy(n_hbm, vn.at[pl.ds(0, C * KCAP)])

        for half in range(2):
            cv = iota + half * L
            cidx = jnp.minimum(cv, C - 1) * KCAP
            h = plsc.load_gather(vsc, [cidx])
            hs[pl.ds(half * L, L)] = jnp.where(cv < C, h, _f32(-1.0))
            nh = plsc.load_gather(vn, [cidx])
            hn[pl.ds(half * L, L)] = jnp.where(cv < C, nh, _i32(0))

        def pinit(c, _):
            hp[c] = _i32(0)
            return 0

        lax.fori_loop(0, 2 * L, pinit, 0)

        def oinit(k, _):
            lv[pl.ds(k * L, L)] = jnp.full((L,), -1, _i32)
            return 0

        lax.fori_loop(0, OPAD // L, oinit, 0)

        def zinit(k, _):
            dv[pl.ds(k * L, L)] = jnp.zeros((L,), _f32)
            return 0

        lax.fori_loop(0, OPAD * 5 // L, zinit, 0)

        def mbody(k, _):
            h1 = hs[pl.ds(0, L)]
            h2 = hs[pl.ds(L, L)]
            m = jnp.maximum(jnp.max(h1), jnp.max(h2))

            @pl.when(m > _f32(0.0))
            def _():
                n1 = hn[pl.ds(0, L)]
                n2 = hn[pl.ds(L, L)]
                fi1 = jnp.where(h1 == m, n1 * C + iota, BIG)
                fi2 = jnp.where(h2 == m, n2 * C + (iota + L), BIG)
                fi = jnp.minimum(jnp.min(fi1), jnp.min(fi2))
                csel = lax.rem(fi, _i32(C))
                p = hp[csel]
                base = csel * KCAP + p
                _sstore(dv, k * 5 + 0, _sload(vx1, base), lane0)
                _sstore(dv, k * 5 + 1, _sload(vy1, base), lane0)
                _sstore(dv, k * 5 + 2, _sload(vx2, base), lane0)
                _sstore(dv, k * 5 + 3, _sload(vy2, base), lane0)
                _sstore(dv, k * 5 + 4, m, lane0)
                _sstore(lv, k, csel, lane0)
                pn = p + 1
                hp[csel] = pn
                pc = jnp.minimum(pn, KCAP - 1)
                nxt = _sload(vsc, csel * KCAP + pc)
                _sstore(hs, csel, jnp.where(pn > KCAP - 1, _f32(-1.0), nxt),
                        lane0)
                _sstore(hn, csel, _sload(vn, csel * KCAP + pc), lane0)

            return 0

        lax.fori_loop(0, MAX_NUM, mbody, 0)

        pltpu.sync_copy(dv, dets_hbm)
        pltpu.sync_copy(lv, labels_hbm)


def kernel(multi_bboxes, multi_scores):
    pad = ((0, 0), (0, NP - N))
    bbt = jnp.pad(multi_bboxes.T, pad)
    st = jnp.pad(multi_scores.T[:C], pad)
    k_sc, k_x1, k_y1, k_x2, k_y2, k_n = _nms_kernel(
        st.reshape(-1), bbt.reshape(-1))
    dets_pad, labels_pad = _merge_kernel(
        k_sc.reshape(-1), k_x1.reshape(-1), k_y1.reshape(-1),
        k_x2.reshape(-1), k_y2.reshape(-1), k_n.reshape(-1))
    dets = dets_pad.reshape(OPAD, 5)[:MAX_NUM]
    labels = labels_pad[:MAX_NUM]
    return dets, labels
